# Initial kernel scaffold; baseline (speedup 1.0000x reference)
import jax
import jax.numpy as jnp
from jax.experimental import pallas as pl


def kernel(x, edge_index, edge_weight, W, b, scalar):
    raise NotImplementedError("write your pallas kernel here")


# speedup vs baseline: 3.0108x; 3.0108x over previous
import functools

import jax
import jax.numpy as jnp
from jax import lax
from jax.experimental import pallas as pl
from jax.experimental.pallas import tpu as pltpu
from jax.experimental.pallas import tpu_sc as plsc

N = 10000
E = 160000
F = 256
FH = 128
EPT = E // 16
CHUNK = 80
NCH = EPT // CHUNK
NP = 10240
NPT = NP // 16
ESLAB = 160
MROWS = 1000


def _mm_body(x_ref, w_ref, b_ref, o_ref):
    o_ref[0] = lax.dot_general(
        x_ref[...], w_ref[...], (((1,), (1,)), ((), ())),
        preferred_element_type=jnp.float32) + b_ref[pl.ds(pl.program_id(0), 1)]


def _linear(x, W, b):
    return pl.pallas_call(
        _mm_body,
        grid=(2, N // MROWS),
        in_specs=[
            pl.BlockSpec((MROWS, F), lambda c, i: (i, 0)),
            pl.BlockSpec((FH, F), lambda c, i: (c, 0)),
            pl.BlockSpec((2, FH), lambda c, i: (0, 0)),
        ],
        out_specs=pl.BlockSpec((1, MROWS, FH), lambda c, i: (c, i, 0)),
        out_shape=jax.ShapeDtypeStruct((2, N, FH), jnp.float32),
    )(x, W, b.reshape(2, FH))


def _make_spmm(table_rows_per_core: int, scale_w: bool):
    TN = table_rows_per_core
    mesh = plsc.VectorSubcoreMesh(core_axis_name="c", subcore_axis_name="s")

    @functools.partial(
        pl.kernel, mesh=mesh,
        out_type=jax.ShapeDtypeStruct((2, NP, FH), jnp.float32),
        compiler_params=pltpu.CompilerParams(
            use_tc_tiling_on_sc=False, needs_layout_passes=False),
        scratch_types=[
            pltpu.VMEM_SHARED((NP, FH), jnp.float32),
            pltpu.VMEM((CHUNK, FH), jnp.float32),
            pltpu.VMEM((ESLAB, FH), jnp.float32),
            pltpu.VMEM((CHUNK,), jnp.int32),
            pltpu.VMEM((CHUNK,), jnp.int32),
            pltpu.VMEM((CHUNK,), jnp.int32),
            pltpu.VMEM((CHUNK,), jnp.float32),
            pltpu.VMEM((16,), jnp.float32),
            pltpu.SemaphoreType.DMA,
            pltpu.SemaphoreType.DMA,
        ],
    )
    def spmm(table, col, dst, ew, scal, out,
             acc, rowsv, ebuf, idxv, colv, dstv, wv, scalv, sem, msem):
        c = lax.axis_index("c")
        s = lax.axis_index("s")
        zero16 = jnp.zeros((16,), jnp.float32)
        pltpu.sync_copy(scal, scalv)
        sv = scalv[pl.ds(0, 16)]

        def zb(e, cc):
            for f in range(FH // 16):
                rowsv[e, pl.ds(f * 16, 16)] = zero16
            return cc
        lax.fori_loop(0, CHUNK, zb, 0)
        for j in range(NPT // CHUNK):
            pltpu.sync_copy(rowsv, acc.at[pl.ds(s * NPT + j * CHUNK, CHUNK)])
        plsc.subcore_barrier()

        cN = c * TN

        def chunk_body(ci, cc):
            e0 = s * EPT + ci * CHUNK
            cp1 = pltpu.async_copy(col.at[pl.ds(e0, CHUNK)], colv, msem)
            cp2 = pltpu.async_copy(dst.at[pl.ds(e0, CHUNK)], dstv, msem)
            cp3 = pltpu.async_copy(ew.at[pl.ds(e0, CHUNK)], wv, msem)
            cp1.wait(); cp2.wait(); cp3.wait()

            cNv = jnp.full((16,), cN, jnp.int32)

            def ib(i, c2):
                idxv[pl.ds(i * 16, 16)] = colv[pl.ds(i * 16, 16)] + cNv
                return c2
            lax.fori_loop(0, CHUNK // 16, ib, 0)
            if scale_w:
                def wb(i, c2):
                    wv[pl.ds(i * 16, 16)] = wv[pl.ds(i * 16, 16)] * sv
                    return c2
                lax.fori_loop(0, CHUNK // 16, wb, 0)
            pltpu.async_copy(table.at[idxv], rowsv, sem).wait()

            def sb(e, c2):
                ws = plsc.load_gather(wv, [jnp.full((16,), e, jnp.int32)])
                for f in range(FH // 16):
                    v = rowsv[e, pl.ds(f * 16, 16)]
                    rowsv[e, pl.ds(f * 16, 16)] = v * ws
                return c2
            lax.fori_loop(0, CHUNK, sb, 0)

            pltpu.sync_copy(rowsv, acc.at[dstv], add=True)
            return cc
        lax.fori_loop(0, NCH, chunk_body, 0)

        plsc.subcore_barrier()
        for k in range(NPT // ESLAB):
            r0 = s * NPT + k * ESLAB
            pltpu.sync_copy(acc.at[pl.ds(r0, ESLAB)], ebuf)

            def eb(r, cc):
                for f in range(FH // 16):
                    v = ebuf[r, pl.ds(f * 16, 16)]
                    ebuf[r, pl.ds(f * 16, 16)] = jnp.where(
                        v > 0, v, jnp.exp(v) - 1.0)
                return cc
            lax.fori_loop(0, ESLAB, eb, 0)
            pltpu.sync_copy(ebuf, out.at[c, pl.ds(r0, ESLAB)])

    return spmm


_spmm_a = _make_spmm(N, False)
_spmm_b = _make_spmm(NP, True)


def kernel(x, edge_index, edge_weight, W, b, scalar):
    dst = edge_index[0].astype(jnp.int32)
    col = edge_index[1].astype(jnp.int32)
    ew = edge_weight.astype(jnp.float32)
    scal16 = jnp.broadcast_to(scalar.astype(jnp.float32), (16,))

    h1 = _linear(x, W, b)
    o1 = _spmm_a(h1.reshape(2 * N, FH), col, dst, ew, scal16)
    o2 = _spmm_b(o1.reshape(2 * NP, FH), col, dst, ew, scal16)
    return o2[:, :N, :].transpose(1, 0, 2).reshape(N, F)


# speedup vs baseline: 3.4809x; 1.1561x over previous
import functools

import jax
import jax.numpy as jnp
from jax import lax
from jax.experimental import pallas as pl
from jax.experimental.pallas import tpu as pltpu
from jax.experimental.pallas import tpu_sc as plsc

N = 10000
E = 160000
F = 256
FH = 128
EPT = E // 16
CHUNK = 80
NCH = EPT // CHUNK
NP = 10240
NPT = NP // 16
ESLAB = 160
MROWS = 1000


def _mm_body(x_ref, w_ref, b_ref, o_ref):
    o_ref[0] = lax.dot_general(
        x_ref[...], w_ref[...], (((1,), (1,)), ((), ())),
        preferred_element_type=jnp.float32) + b_ref[pl.ds(pl.program_id(0), 1)]


def _linear(x, W, b):
    return pl.pallas_call(
        _mm_body,
        grid=(2, N // MROWS),
        in_specs=[
            pl.BlockSpec((MROWS, F), lambda c, i: (i, 0)),
            pl.BlockSpec((FH, F), lambda c, i: (c, 0)),
            pl.BlockSpec((2, FH), lambda c, i: (0, 0)),
        ],
        out_specs=pl.BlockSpec((1, MROWS, FH), lambda c, i: (c, i, 0)),
        out_shape=jax.ShapeDtypeStruct((2, N, FH), jnp.float32),
    )(x, W, b.reshape(2, FH))


def _make_spmm(table_rows_per_core: int, scale_w: bool):
    TN = table_rows_per_core
    mesh = plsc.VectorSubcoreMesh(core_axis_name="c", subcore_axis_name="s")

    @functools.partial(
        pl.kernel, mesh=mesh,
        out_type=jax.ShapeDtypeStruct((2, NP, FH), jnp.float32),
        compiler_params=pltpu.CompilerParams(
            use_tc_tiling_on_sc=False, needs_layout_passes=False),
        scratch_types=[
            pltpu.VMEM_SHARED((NP, FH), jnp.float32),
            pltpu.VMEM((2, CHUNK, FH), jnp.float32),
            pltpu.VMEM((ESLAB, FH), jnp.float32),
            pltpu.VMEM((2, CHUNK), jnp.int32),
            pltpu.VMEM((2, CHUNK), jnp.int32),
            pltpu.VMEM((2, CHUNK), jnp.int32),
            pltpu.VMEM((2, CHUNK), jnp.float32),
            pltpu.VMEM((16,), jnp.float32),
            pltpu.SemaphoreType.DMA,
            pltpu.SemaphoreType.DMA,
        ],
    )
    def spmm(table, col, dst, ew, scal, out,
             acc, rowsv, ebuf, idxv, colv, dstv, wv, scalv, sem, msem):
        c = lax.axis_index("c")
        s = lax.axis_index("s")
        zero16 = jnp.zeros((16,), jnp.float32)
        pltpu.sync_copy(scal, scalv)
        sv = scalv[pl.ds(0, 16)]

        def zb(e, cc):
            for f in range(FH // 16):
                rowsv[0, e, pl.ds(f * 16, 16)] = zero16
            return cc
        lax.fori_loop(0, CHUNK, zb, 0)
        for j in range(NPT // CHUNK):
            pltpu.sync_copy(rowsv.at[0],
                            acc.at[pl.ds(s * NPT + j * CHUNK, CHUNK)])
        plsc.subcore_barrier()

        cN = c * TN
        cNv = jnp.full((16,), cN, jnp.int32)
        ebase = s * EPT

        def stage_meta(e0, p):
            cp1 = pltpu.async_copy(col.at[pl.ds(e0, CHUNK)], colv.at[p], msem)
            cp2 = pltpu.async_copy(dst.at[pl.ds(e0, CHUNK)], dstv.at[p], msem)
            cp3 = pltpu.async_copy(ew.at[pl.ds(e0, CHUNK)], wv.at[p], msem)
            return cp1, cp2, cp3

        def build_idx(p):
            def ib(i, c2):
                idxv[p, pl.ds(i * 16, 16)] = colv[p, pl.ds(i * 16, 16)] + cNv
                return c2
            lax.fori_loop(0, CHUNK // 16, ib, 0)
            if scale_w:
                def wb(i, c2):
                    wv[p, pl.ds(i * 16, 16)] = wv[p, pl.ds(i * 16, 16)] * sv
                    return c2
                lax.fori_loop(0, CHUNK // 16, wb, 0)

        def fire_gather(p):
            return pltpu.async_copy(table.at[idxv.at[p]], rowsv.at[p], sem)

        def wait_gather(p):
            pltpu.make_async_copy(table.at[idxv.at[p]], rowsv.at[p], sem).wait()

        pconst = [jnp.full((16,), pp, jnp.int32) for pp in range(2)]

        def scale_scatter(p):
            def sb(e, c2):
                ws = plsc.load_gather(
                    wv, [pconst[p], jnp.full((16,), e, jnp.int32)])
                for f in range(FH // 16):
                    v = rowsv[p, e, pl.ds(f * 16, 16)]
                    rowsv[p, e, pl.ds(f * 16, 16)] = v * ws
                return c2
            lax.fori_loop(0, CHUNK, sb, 0)
            pltpu.sync_copy(rowsv.at[p], acc.at[dstv.at[p]], add=True)

        for cp in stage_meta(ebase, 0):
            cp.wait()
        build_idx(0)
        fire_gather(0)

        def half(g, p):
            mcs = stage_meta(ebase + (g + 1) * CHUNK, 1 - p)
            wait_gather(p)
            scale_scatter(p)
            for cp in mcs:
                cp.wait()
            build_idx(1 - p)
            fire_gather(1 - p)

        def pair_body(i, cc):
            half(2 * i, 0)
            half(2 * i + 1, 1)
            return cc
        lax.fori_loop(0, (NCH - 1) // 2, pair_body, 0)

        wait_gather(0)
        scale_scatter(0)

        plsc.subcore_barrier()
        for k in range(NPT // ESLAB):
            r0 = s * NPT + k * ESLAB
            pltpu.sync_copy(acc.at[pl.ds(r0, ESLAB)], ebuf)

            def eb(r, cc):
                for f in range(FH // 16):
                    v = ebuf[r, pl.ds(f * 16, 16)]
                    ebuf[r, pl.ds(f * 16, 16)] = jnp.where(
                        v > 0, v, jnp.exp(v) - 1.0)
                return cc
            lax.fori_loop(0, ESLAB, eb, 0)
            pltpu.sync_copy(ebuf, out.at[c, pl.ds(r0, ESLAB)])

    return spmm


_spmm_a = _make_spmm(N, False)
_spmm_b = _make_spmm(NP, True)


def kernel(x, edge_index, edge_weight, W, b, scalar):
    dst = edge_index[0].astype(jnp.int32)
    col = edge_index[1].astype(jnp.int32)
    ew = edge_weight.astype(jnp.float32)
    scal16 = jnp.broadcast_to(scalar.astype(jnp.float32), (16,))

    h1 = _linear(x, W, b)
    o1 = _spmm_a(h1.reshape(2 * N, FH), col, dst, ew, scal16)
    o2 = _spmm_b(o1.reshape(2 * NP, FH), col, dst, ew, scal16)
    return o2[:, :N, :].transpose(1, 0, 2).reshape(N, F)


# speedup vs baseline: 5.3455x; 1.5357x over previous
import functools

import jax
import jax.numpy as jnp
from jax import lax
from jax.experimental import pallas as pl
from jax.experimental.pallas import tpu as pltpu
from jax.experimental.pallas import tpu_sc as plsc

N = 10000
E = 160000
F = 256
FH = 128
EPT = E // 16
CHUNK = 80
NCH = EPT // CHUNK
NP = 10240
NPT = NP // 16
ESLAB = 160
MROWS = 1000


def _mm_body(x_ref, w_ref, b_ref, o_ref):
    o_ref[0] = lax.dot_general(
        x_ref[...], w_ref[...], (((1,), (1,)), ((), ())),
        preferred_element_type=jnp.float32) + b_ref[pl.ds(pl.program_id(0), 1)]


def _linear(x, W, b):
    return pl.pallas_call(
        _mm_body,
        grid=(2, N // MROWS),
        in_specs=[
            pl.BlockSpec((MROWS, F), lambda c, i: (i, 0)),
            pl.BlockSpec((FH, F), lambda c, i: (c, 0)),
            pl.BlockSpec((2, FH), lambda c, i: (0, 0)),
        ],
        out_specs=pl.BlockSpec((1, MROWS, FH), lambda c, i: (c, i, 0)),
        out_shape=jax.ShapeDtypeStruct((2, N, FH), jnp.float32),
    )(x, W, b.reshape(2, FH))


def _make_spmm(table_rows_per_core: int, scale_w: bool):
    TN = table_rows_per_core
    mesh = plsc.VectorSubcoreMesh(core_axis_name="c", subcore_axis_name="s")

    @functools.partial(
        pl.kernel, mesh=mesh,
        out_type=jax.ShapeDtypeStruct((2, NP, FH), jnp.float32),
        compiler_params=pltpu.CompilerParams(
            use_tc_tiling_on_sc=False, needs_layout_passes=False),
        scratch_types=[
            pltpu.VMEM_SHARED((NP, FH), jnp.float32),
            pltpu.VMEM((2, CHUNK, FH), jnp.float32),
            pltpu.VMEM((ESLAB, FH), jnp.float32),
            pltpu.VMEM((2, CHUNK), jnp.int32),
            pltpu.VMEM((2, CHUNK), jnp.int32),
            pltpu.VMEM((2, CHUNK), jnp.int32),
            pltpu.VMEM((2, CHUNK), jnp.float32),
            pltpu.VMEM((16,), jnp.float32),
            pltpu.SemaphoreType.DMA,
            pltpu.SemaphoreType.DMA,
        ],
    )
    def spmm(table, col, dst, ew, scal, out,
             acc, rowsv, ebuf, idxv, colv, dstv, wv, scalv, sem, msem):
        c = lax.axis_index("c")
        s = lax.axis_index("s")
        zero16 = jnp.zeros((16,), jnp.float32)
        pltpu.sync_copy(scal, scalv)
        sv = scalv[pl.ds(0, 16)]

        def zb(e, cc):
            for f in range(FH // 16):
                rowsv[0, e, pl.ds(f * 16, 16)] = zero16
            return cc
        lax.fori_loop(0, CHUNK, zb, 0)
        for j in range(NPT // CHUNK):
            pltpu.sync_copy(rowsv.at[0],
                            acc.at[pl.ds(s * NPT + j * CHUNK, CHUNK)])
        plsc.subcore_barrier()

        cN = c * TN
        cNv = jnp.full((16,), cN, jnp.int32)
        ebase = s * EPT

        def stage_meta(e0, p):
            cp1 = pltpu.async_copy(col.at[pl.ds(e0, CHUNK)], colv.at[p], msem)
            cp2 = pltpu.async_copy(dst.at[pl.ds(e0, CHUNK)], dstv.at[p], msem)
            cp3 = pltpu.async_copy(ew.at[pl.ds(e0, CHUNK)], wv.at[p], msem)
            return cp1, cp2, cp3

        def build_idx(p):
            def ib(i, c2):
                idxv[p, pl.ds(i * 16, 16)] = colv[p, pl.ds(i * 16, 16)] + cNv
                return c2
            lax.fori_loop(0, CHUNK // 16, ib, 0)
            if scale_w:
                def wb(i, c2):
                    wv[p, pl.ds(i * 16, 16)] = wv[p, pl.ds(i * 16, 16)] * sv
                    return c2
                lax.fori_loop(0, CHUNK // 16, wb, 0)

        def fire_gather(p):
            return pltpu.async_copy(table.at[idxv.at[p]], rowsv.at[p], sem)

        def wait_gather(p):
            pltpu.make_async_copy(table.at[idxv.at[p]], rowsv.at[p], sem).wait()

        pconst = [jnp.full((16,), pp, jnp.int32) for pp in range(2)]

        def scale_scatter(p):
            @plsc.parallel_loop(0, CHUNK, 2, unroll=2)
            def sb(e):
                for k in range(2):
                    ws = plsc.load_gather(
                        wv, [pconst[p], jnp.full((16,), e + k, jnp.int32)])
                    for f in range(FH // 16):
                        v = rowsv[p, e + k, pl.ds(f * 16, 16)]
                        rowsv[p, e + k, pl.ds(f * 16, 16)] = v * ws
            pltpu.sync_copy(rowsv.at[p], acc.at[dstv.at[p]], add=True)

        emax = E - CHUNK

        def wait_meta(p):
            for r in (colv, dstv):
                pltpu.make_async_copy(col.at[pl.ds(0, CHUNK)],
                                      r.at[p], msem).wait()
            pltpu.make_async_copy(ew.at[pl.ds(0, CHUNK)],
                                  wv.at[p], msem).wait()

        for cp in stage_meta(ebase, 0):
            cp.wait()
        build_idx(0)
        fire_gather(0)
        stage_meta(ebase + CHUNK, 1)

        def half(g, p):
            wait_meta(1 - p)
            build_idx(1 - p)
            wait_gather(p)
            fire_gather(1 - p)
            scale_scatter(p)
            stage_meta(jnp.minimum(ebase + (g + 2) * CHUNK, emax), p)

        def pair_body(i, cc):
            half(2 * i, 0)
            half(2 * i + 1, 1)
            return cc
        lax.fori_loop(0, (NCH - 1) // 2, pair_body, 0)

        wait_meta(1)
        wait_gather(0)
        scale_scatter(0)

        plsc.subcore_barrier()
        for k in range(NPT // ESLAB):
            r0 = s * NPT + k * ESLAB
            pltpu.sync_copy(acc.at[pl.ds(r0, ESLAB)], ebuf)

            def eb(r, cc):
                for f in range(FH // 16):
                    v = ebuf[r, pl.ds(f * 16, 16)]
                    ebuf[r, pl.ds(f * 16, 16)] = jnp.where(
                        v > 0, v, jnp.exp(v) - 1.0)
                return cc
            lax.fori_loop(0, ESLAB, eb, 0)
            pltpu.sync_copy(ebuf, out.at[c, pl.ds(r0, ESLAB)])

    return spmm


_spmm_a = _make_spmm(N, False)
_spmm_b = _make_spmm(NP, True)


def kernel(x, edge_index, edge_weight, W, b, scalar):
    dst = edge_index[0].astype(jnp.int32)
    col = edge_index[1].astype(jnp.int32)
    ew = edge_weight.astype(jnp.float32)
    scal16 = jnp.broadcast_to(scalar.astype(jnp.float32), (16,))

    h1 = _linear(x, W, b)
    o1 = _spmm_a(h1.reshape(2 * N, FH), col, dst, ew, scal16)
    o2 = _spmm_b(o1.reshape(2 * NP, FH), col, dst, ew, scal16)
    return o2[:, :N, :].transpose(1, 0, 2).reshape(N, F)


# speedup vs baseline: 6.0926x; 1.1398x over previous
import functools

import jax
import jax.numpy as jnp
from jax import lax
from jax.experimental import pallas as pl
from jax.experimental.pallas import tpu as pltpu
from jax.experimental.pallas import tpu_sc as plsc

N = 10000
E = 160000
F = 256
FH = 128
EPT = E // 16
CHUNK = 80
NCH = EPT // CHUNK
NP = 10240
NPT = NP // 16
ESLAB = 160
MROWS = 1000


def _mm_body(x_ref, w_ref, b_ref, o_ref):
    o_ref[0] = lax.dot_general(
        x_ref[...], w_ref[...], (((1,), (1,)), ((), ())),
        preferred_element_type=jnp.float32) + b_ref[pl.ds(pl.program_id(0), 1)]


def _linear(x, W, b):
    return pl.pallas_call(
        _mm_body,
        grid=(2, N // MROWS),
        in_specs=[
            pl.BlockSpec((MROWS, F), lambda c, i: (i, 0)),
            pl.BlockSpec((FH, F), lambda c, i: (c, 0)),
            pl.BlockSpec((2, FH), lambda c, i: (0, 0)),
        ],
        out_specs=pl.BlockSpec((1, MROWS, FH), lambda c, i: (c, i, 0)),
        out_shape=jax.ShapeDtypeStruct((2, N, FH), jnp.float32),
    )(x, W, b.reshape(2, FH))


def _make_spmm(table_rows_per_core: int, scale_w: bool):
    TN = table_rows_per_core
    mesh = plsc.VectorSubcoreMesh(core_axis_name="c", subcore_axis_name="s")

    @functools.partial(
        pl.kernel, mesh=mesh,
        out_type=jax.ShapeDtypeStruct((2, NP, FH), jnp.float32),
        compiler_params=pltpu.CompilerParams(
            use_tc_tiling_on_sc=False, needs_layout_passes=False),
        scratch_types=[
            pltpu.VMEM_SHARED((NP, FH), jnp.float32),
            pltpu.VMEM((2, CHUNK, FH), jnp.float32),
            pltpu.VMEM((ESLAB, FH), jnp.float32),
            pltpu.VMEM((2, CHUNK), jnp.int32),
            pltpu.VMEM((4, CHUNK), jnp.int32),
            pltpu.VMEM((4, CHUNK), jnp.int32),
            pltpu.VMEM((4, CHUNK), jnp.float32),
            pltpu.VMEM((16,), jnp.float32),
            pltpu.SemaphoreType.DMA,
            pltpu.SemaphoreType.DMA,
            pltpu.SemaphoreType.DMA,
        ],
    )
    def spmm(table, col, dst, ew, scal, out,
             acc, rowsv, ebuf, idxv, colv, dstv, wv, scalv, sem, msem, ssem):
        c = lax.axis_index("c")
        s = lax.axis_index("s")
        zero16 = jnp.zeros((16,), jnp.float32)
        pltpu.sync_copy(scal, scalv)
        sv = scalv[pl.ds(0, 16)]

        def zb(e, cc):
            for f in range(FH // 16):
                rowsv[0, e, pl.ds(f * 16, 16)] = zero16
            return cc
        lax.fori_loop(0, CHUNK, zb, 0)
        for j in range(NPT // CHUNK):
            pltpu.sync_copy(rowsv.at[0],
                            acc.at[pl.ds(s * NPT + j * CHUNK, CHUNK)])
        plsc.subcore_barrier()

        cN = c * TN
        cNv = jnp.full((16,), cN, jnp.int32)
        ebase = s * EPT

        def stage_meta(e0, p):
            cp1 = pltpu.async_copy(col.at[pl.ds(e0, CHUNK)], colv.at[p], msem)
            cp2 = pltpu.async_copy(dst.at[pl.ds(e0, CHUNK)], dstv.at[p], msem)
            cp3 = pltpu.async_copy(ew.at[pl.ds(e0, CHUNK)], wv.at[p], msem)
            return cp1, cp2, cp3

        def build_idx(pi, q):
            def ib(i, c2):
                idxv[pi, pl.ds(i * 16, 16)] = colv[q, pl.ds(i * 16, 16)] + cNv
                return c2
            lax.fori_loop(0, CHUNK // 16, ib, 0)
            if scale_w:
                def wb(i, c2):
                    wv[q, pl.ds(i * 16, 16)] = wv[q, pl.ds(i * 16, 16)] * sv
                    return c2
                lax.fori_loop(0, CHUNK // 16, wb, 0)

        def fire_gather(p):
            return pltpu.async_copy(table.at[idxv.at[p]], rowsv.at[p], sem)

        def wait_gather(p):
            pltpu.make_async_copy(table.at[idxv.at[p]], rowsv.at[p], sem).wait()

        pconst = [jnp.full((16,), pp, jnp.int32) for pp in range(4)]

        def scale(p, q):
            @plsc.parallel_loop(0, CHUNK, 2, unroll=2)
            def sb(e):
                for k in range(2):
                    ws = plsc.load_gather(
                        wv, [pconst[q], jnp.full((16,), e + k, jnp.int32)])
                    for f in range(FH // 16):
                        v = rowsv[p, e + k, pl.ds(f * 16, 16)]
                        rowsv[p, e + k, pl.ds(f * 16, 16)] = v * ws

        def fire_scatter(p, q):
            return pltpu.async_copy(rowsv.at[p], acc.at[dstv.at[q]],
                                    ssem, add=True)

        def wait_scatter(p, q):
            pltpu.make_async_copy(rowsv.at[p], acc.at[dstv.at[q]],
                                  ssem).wait()

        emax = E - CHUNK

        def wait_meta(q):
            for r in (colv, dstv):
                pltpu.make_async_copy(col.at[pl.ds(0, CHUNK)],
                                      r.at[q], msem).wait()
            pltpu.make_async_copy(ew.at[pl.ds(0, CHUNK)],
                                  wv.at[q], msem).wait()

        for cp in stage_meta(ebase, 0):
            cp.wait()
        build_idx(0, 0)
        fire_gather(0)
        stage_meta(ebase + CHUNK, 1)

        def half(i, g, j):
            p = j % 2
            wait_meta((j + 1) % 4)
            build_idx(1 - p, (j + 1) % 4)
            wait_gather(p)
            if j == 0:
                @pl.when(i > 0)
                def _():
                    wait_scatter(1, 3)
            else:
                wait_scatter(1 - p, j - 1)
            fire_gather(1 - p)
            scale(p, j)
            fire_scatter(p, j)
            stage_meta(jnp.minimum(ebase + (g + 2) * CHUNK, emax),
                       (j + 2) % 4)

        def quad_body(i, cc):
            g = 4 * i
            for j in range(4):
                half(i, g + j, j)
            return cc
        lax.fori_loop(0, (NCH - 1) // 4, quad_body, 0)

        wait_meta(1)
        wait_gather(0)
        wait_scatter(1, 3)
        scale(0, 0)
        pltpu.sync_copy(rowsv.at[0], acc.at[dstv.at[0]], add=True)

        plsc.subcore_barrier()
        for k in range(NPT // ESLAB):
            r0 = s * NPT + k * ESLAB
            pltpu.sync_copy(acc.at[pl.ds(r0, ESLAB)], ebuf)

            def eb(r, cc):
                for f in range(FH // 16):
                    v = ebuf[r, pl.ds(f * 16, 16)]
                    ebuf[r, pl.ds(f * 16, 16)] = jnp.where(
                        v > 0, v, jnp.exp(v) - 1.0)
                return cc
            lax.fori_loop(0, ESLAB, eb, 0)
            pltpu.sync_copy(ebuf, out.at[c, pl.ds(r0, ESLAB)])

    return spmm


_spmm_a = _make_spmm(N, False)
_spmm_b = _make_spmm(NP, True)


def kernel(x, edge_index, edge_weight, W, b, scalar):
    dst = edge_index[0].astype(jnp.int32)
    col = edge_index[1].astype(jnp.int32)
    ew = edge_weight.astype(jnp.float32)
    scal16 = jnp.broadcast_to(scalar.astype(jnp.float32), (16,))

    h1 = _linear(x, W, b)
    o1 = _spmm_a(h1.reshape(2 * N, FH), col, dst, ew, scal16)
    o2 = _spmm_b(o1.reshape(2 * NP, FH), col, dst, ew, scal16)
    return o2[:, :N, :].transpose(1, 0, 2).reshape(N, F)


# speedup vs baseline: 6.1610x; 1.0112x over previous
import functools

import jax
import jax.numpy as jnp
from jax import lax
from jax.experimental import pallas as pl
from jax.experimental.pallas import tpu as pltpu
from jax.experimental.pallas import tpu_sc as plsc

N = 10000
E = 160000
F = 256
FH = 128
EPT = E // 16
CHUNK = 80
NCH = EPT // CHUNK
NP = 10240
NPT = NP // 16
ESLAB = 80
MROWS = 1000


def _mm_body(x_ref, w_ref, b_ref, o_ref):
    o_ref[0] = lax.dot_general(
        x_ref[...], w_ref[...], (((1,), (1,)), ((), ())),
        preferred_element_type=jnp.float32) + b_ref[pl.ds(pl.program_id(0), 1)]


def _linear(x, W, b):
    return pl.pallas_call(
        _mm_body,
        grid=(2, N // MROWS),
        in_specs=[
            pl.BlockSpec((MROWS, F), lambda c, i: (i, 0)),
            pl.BlockSpec((FH, F), lambda c, i: (c, 0)),
            pl.BlockSpec((2, FH), lambda c, i: (0, 0)),
        ],
        out_specs=pl.BlockSpec((1, MROWS, FH), lambda c, i: (c, i, 0)),
        out_shape=jax.ShapeDtypeStruct((2, N, FH), jnp.float32),
    )(x, W, b.reshape(2, FH))


def _make_spmm(table_rows_per_core: int, scale_w: bool):
    TN = table_rows_per_core
    mesh = plsc.VectorSubcoreMesh(core_axis_name="c", subcore_axis_name="s")

    @functools.partial(
        pl.kernel, mesh=mesh,
        out_type=jax.ShapeDtypeStruct((2, NP, FH), jnp.float32),
        compiler_params=pltpu.CompilerParams(
            use_tc_tiling_on_sc=False, needs_layout_passes=False),
        scratch_types=[
            pltpu.VMEM_SHARED((NP, FH), jnp.float32),
            pltpu.VMEM((2, CHUNK, FH), jnp.float32),
            pltpu.VMEM((2, ESLAB, FH), jnp.float32),
            pltpu.VMEM((2, CHUNK), jnp.int32),
            pltpu.VMEM((4, CHUNK), jnp.int32),
            pltpu.VMEM((4, CHUNK), jnp.int32),
            pltpu.VMEM((4, CHUNK), jnp.float32),
            pltpu.VMEM((16,), jnp.float32),
            pltpu.SemaphoreType.DMA,
            pltpu.SemaphoreType.DMA,
            pltpu.SemaphoreType.DMA,
        ],
    )
    def spmm(table, col, dst, ew, scal, out,
             acc, rowsv, ebuf, idxv, colv, dstv, wv, scalv, sem, msem, ssem):
        c = lax.axis_index("c")
        s = lax.axis_index("s")
        zero16 = jnp.zeros((16,), jnp.float32)
        pltpu.sync_copy(scal, scalv)
        sv = scalv[pl.ds(0, 16)]

        def zb(e, cc):
            for f in range(FH // 16):
                rowsv[0, e, pl.ds(f * 16, 16)] = zero16
            return cc
        lax.fori_loop(0, CHUNK, zb, 0)
        for j in range(NPT // CHUNK):
            pltpu.sync_copy(rowsv.at[0],
                            acc.at[pl.ds(s * NPT + j * CHUNK, CHUNK)])
        plsc.subcore_barrier()

        cN = c * TN
        cNv = jnp.full((16,), cN, jnp.int32)
        ebase = s * EPT

        def stage_meta(e0, p):
            cp1 = pltpu.async_copy(col.at[pl.ds(e0, CHUNK)], colv.at[p], msem)
            cp2 = pltpu.async_copy(dst.at[pl.ds(e0, CHUNK)], dstv.at[p], msem)
            cp3 = pltpu.async_copy(ew.at[pl.ds(e0, CHUNK)], wv.at[p], msem)
            return cp1, cp2, cp3

        def build_idx(pi, q):
            def ib(i, c2):
                idxv[pi, pl.ds(i * 16, 16)] = colv[q, pl.ds(i * 16, 16)] + cNv
                return c2
            lax.fori_loop(0, CHUNK // 16, ib, 0)
            if scale_w:
                def wb(i, c2):
                    wv[q, pl.ds(i * 16, 16)] = wv[q, pl.ds(i * 16, 16)] * sv
                    return c2
                lax.fori_loop(0, CHUNK // 16, wb, 0)

        def fire_gather(p):
            return pltpu.async_copy(table.at[idxv.at[p]], rowsv.at[p], sem)

        def wait_gather(p):
            pltpu.make_async_copy(table.at[idxv.at[p]], rowsv.at[p], sem).wait()

        pconst = [jnp.full((16,), pp, jnp.int32) for pp in range(4)]

        def scale(p, q):
            @plsc.parallel_loop(0, CHUNK, 2, unroll=2)
            def sb(e):
                for k in range(2):
                    ws = plsc.load_gather(
                        wv, [pconst[q], jnp.full((16,), e + k, jnp.int32)])
                    for f in range(FH // 16):
                        v = rowsv[p, e + k, pl.ds(f * 16, 16)]
                        rowsv[p, e + k, pl.ds(f * 16, 16)] = v * ws

        def fire_scatter(p, q):
            return pltpu.async_copy(rowsv.at[p], acc.at[dstv.at[q]],
                                    ssem, add=True)

        def wait_scatter(p, q):
            pltpu.make_async_copy(rowsv.at[p], acc.at[dstv.at[q]],
                                  ssem).wait()

        emax = E - CHUNK

        def wait_meta(q):
            for r in (colv, dstv):
                pltpu.make_async_copy(col.at[pl.ds(0, CHUNK)],
                                      r.at[q], msem).wait()
            pltpu.make_async_copy(ew.at[pl.ds(0, CHUNK)],
                                  wv.at[q], msem).wait()

        for cp in stage_meta(ebase, 0):
            cp.wait()
        build_idx(0, 0)
        fire_gather(0)
        stage_meta(ebase + CHUNK, 1)

        def half(i, g, j):
            p = j % 2
            wait_meta((j + 1) % 4)
            build_idx(1 - p, (j + 1) % 4)
            wait_gather(p)
            if j == 0:
                @pl.when(i > 0)
                def _():
                    wait_scatter(1, 3)
            else:
                wait_scatter(1 - p, j - 1)
            fire_gather(1 - p)
            scale(p, j)
            fire_scatter(p, j)
            stage_meta(jnp.minimum(ebase + (g + 2) * CHUNK, emax),
                       (j + 2) % 4)

        def quad_body(i, cc):
            g = 4 * i
            for j in range(4):
                half(i, g + j, j)
            return cc
        lax.fori_loop(0, (NCH - 1) // 4, quad_body, 0)

        wait_meta(1)
        wait_gather(0)
        wait_scatter(1, 3)
        scale(0, 0)
        pltpu.sync_copy(rowsv.at[0], acc.at[dstv.at[0]], add=True)

        plsc.subcore_barrier()
        nslab = NPT // ESLAB
        rbase = s * NPT
        pltpu.async_copy(acc.at[pl.ds(rbase, ESLAB)], ebuf.at[0], msem)
        outcps = []
        for k in range(nslab):
            ep = k % 2
            pltpu.make_async_copy(acc.at[pl.ds(rbase, ESLAB)],
                                  ebuf.at[ep], msem).wait()
            if k + 1 < nslab:
                if k + 1 >= 2:
                    outcps[k - 1].wait()
                pltpu.async_copy(
                    acc.at[pl.ds(rbase + (k + 1) * ESLAB, ESLAB)],
                    ebuf.at[1 - ep], msem)

            def eb(r, cc, ep=ep):
                for f in range(FH // 16):
                    v = ebuf[ep, r, pl.ds(f * 16, 16)]
                    ebuf[ep, r, pl.ds(f * 16, 16)] = jnp.where(
                        v > 0, v, jnp.exp(v) - 1.0)
                return cc
            lax.fori_loop(0, ESLAB, eb, 0)
            outcps.append(pltpu.async_copy(
                ebuf.at[ep], out.at[c, pl.ds(rbase + k * ESLAB, ESLAB)],
                ssem))
        for cp in outcps[-2:]:
            cp.wait()

    return spmm


_spmm_a = _make_spmm(N, False)
_spmm_b = _make_spmm(NP, True)


def kernel(x, edge_index, edge_weight, W, b, scalar):
    dst = edge_index[0].astype(jnp.int32)
    col = edge_index[1].astype(jnp.int32)
    ew = edge_weight.astype(jnp.float32)
    scal16 = jnp.broadcast_to(scalar.astype(jnp.float32), (16,))

    h1 = _linear(x, W, b)
    o1 = _spmm_a(h1.reshape(2 * N, FH), col, dst, ew, scal16)
    o2 = _spmm_b(o1.reshape(2 * NP, FH), col, dst, ew, scal16)
    return o2[:, :N, :].transpose(1, 0, 2).reshape(N, F)
